# Initial kernel scaffold; baseline (speedup 1.0000x reference)
import jax
import jax.numpy as jnp
from jax.experimental import pallas as pl


def kernel(z, embedding):
    raise NotImplementedError("write your pallas kernel here")


# speedup vs baseline: 13.9641x; 13.9641x over previous
import functools

import jax
import jax.numpy as jnp
from jax.experimental import pallas as pl
from jax.experimental.pallas import tpu as pltpu

_NUM_EMBEDDINGS = 8192
_EMB_DIM = 32
_COMMITMENT_COST = 0.25
_TOKEN_TILE = 512


def _vq_body(z_ref, e_ref, out_ref, loss_ref, *, scale):
    b = pl.program_id(0)
    t = pl.program_id(1)
    nb = pl.num_programs(0)
    nt = pl.num_programs(1)

    zt = z_ref[0]
    emb = e_ref[...]
    ztT = zt.T

    mm = jnp.dot(ztT, emb, preferred_element_type=jnp.float32)
    z_sq = jnp.sum(ztT * ztT, axis=1, keepdims=True)
    e_sq = jnp.sum(emb * emb, axis=0, keepdims=True)
    dist = (z_sq - 2.0 * mm) + e_sq

    tile_t = dist.shape[0]
    idx = jnp.argmin(dist, axis=1).astype(jnp.int32)

    sub = jax.lax.broadcasted_iota(jnp.int32, (_NUM_EMBEDDINGS, tile_t), 0)
    one_hot = (sub == idx[None, :]).astype(jnp.float32)
    q = jnp.dot(emb, one_hot, preferred_element_type=jnp.float32)

    diff = q - zt
    out_ref[0] = zt + diff
    part = jnp.sum(diff * diff).reshape(1, 1)

    first = jnp.logical_and(b == 0, t == 0)
    last = jnp.logical_and(b == nb - 1, t == nt - 1)

    @pl.when(first)
    def _():
        loss_ref[...] = jnp.zeros((1, 1), jnp.float32)

    loss_ref[...] += part

    @pl.when(last)
    def _():
        loss_ref[...] = loss_ref[...] * jnp.float32(scale)


def kernel(z, embedding):
    bsz, cdim, h, w = z.shape
    hw = h * w
    z3 = z.reshape(bsz, cdim, hw)
    n_tok = bsz * hw
    tile = _TOKEN_TILE
    grid = (bsz, hw // tile)
    scale = (1.0 + _COMMITMENT_COST) / float(n_tok * cdim)

    body = functools.partial(_vq_body, scale=scale)

    out, loss = pl.pallas_call(
        body,
        grid=grid,
        in_specs=[
            pl.BlockSpec((1, cdim, tile), lambda b, t: (b, 0, t)),
            pl.BlockSpec((cdim, _NUM_EMBEDDINGS), lambda b, t: (0, 0)),
        ],
        out_specs=[
            pl.BlockSpec((1, cdim, tile), lambda b, t: (b, 0, t)),
            pl.BlockSpec((1, 1), lambda b, t: (0, 0)),
        ],
        out_shape=[
            jax.ShapeDtypeStruct((bsz, cdim, hw), jnp.float32),
            jax.ShapeDtypeStruct((1, 1), jnp.float32),
        ],
        compiler_params=pltpu.CompilerParams(
            dimension_semantics=("arbitrary", "arbitrary"),
        ),
    )(z3, embedding)

    return out.reshape(z.shape), loss[0, 0]
